# baseline (speedup 1.0000x reference)
import functools

import jax
import jax.numpy as jnp
import numpy as np
from jax.experimental import pallas as pl
from jax.experimental.pallas import tpu as pltpu

_DIM = 32
_OUT = 32


def _blade_combine(a, b):
    if a == 0:
        return (b, 1)
    if b == 0:
        return (a, 1)
    c = a ^ b
    s = 1
    p = max(a, b)
    d = bin(a).count('1')
    e = 1
    while e <= p:
        if e & a:
            d -= 1
        if d & 1 and e & b:
            s = -s
        e *= 2
    return (c, s)


def _tables():
    dim, out = _DIM, _OUT
    rs = np.zeros((dim, dim), dtype=np.float32)
    for i in range(dim):
        for j in range(dim):
            _, s = _blade_combine(i, j)
            rs[i, j] = s
    ts = np.zeros((dim, dim, out, dim), dtype=np.float32)
    for i in range(dim):
        for j in range(dim):
            ts[j, i, :, i ^ j] = rs[i, j]
    ts = ts.reshape(dim, dim, out * dim)
    ed = np.kron(np.eye(out, dtype=np.float32), np.ones((1, dim), np.float32))
    return ts, ed


_TS, _ED = _tables()


def _body(x_ref, w_ref, ts_ref, ed_ref, b_ref, o_ref):
    wb = jax.lax.dot_general(
        w_ref[...], ed_ref[...], (((0,), (0,)), ((), ())),
        preferred_element_type=jnp.float32)
    g = jnp.sum(ts_ref[...] * wb[:, None, :], axis=0)
    acc = jax.lax.dot_general(
        x_ref[...].astype(jnp.bfloat16), g.astype(jnp.bfloat16),
        (((1,), (0,)), ((), ())),
        preferred_element_type=jnp.float32) + b_ref[...]
    o_ref[...] = acc.astype(jnp.bfloat16)


@functools.partial(jax.jit, static_argnames=())
def kernel(input, weight, bias):
    batch = input.shape[0]
    dim, out = _DIM, _OUT
    block = 4096
    bflat = jnp.tile(bias, out)[None, :]
    flat = pl.pallas_call(
        _body,
        grid=(batch // block,),
        in_specs=[
            pl.BlockSpec((block, dim), lambda i: (i, 0)),
            pl.BlockSpec((out, dim), lambda i: (0, 0)),
            pl.BlockSpec((dim, dim, out * dim), lambda i: (0, 0, 0)),
            pl.BlockSpec((out, out * dim), lambda i: (0, 0)),
            pl.BlockSpec((1, out * dim), lambda i: (0, 0)),
        ],
        out_specs=pl.BlockSpec((block, out * dim), lambda i: (i, 0)),
        out_shape=jax.ShapeDtypeStruct((batch, out * dim), jnp.bfloat16),
        compiler_params=pltpu.CompilerParams(
            dimension_semantics=("parallel",)),
    )(input, weight, jnp.asarray(_TS), jnp.asarray(_ED), bflat)
    return flat.astype(jnp.float32).reshape(batch, out, dim)


# speedup vs baseline: 1.0113x; 1.0113x over previous
import functools

import jax
import jax.numpy as jnp
import numpy as np
from jax.experimental import pallas as pl
from jax.experimental.pallas import tpu as pltpu

_DIM = 32
_OUT = 32


def _blade_combine(a, b):
    if a == 0:
        return (b, 1)
    if b == 0:
        return (a, 1)
    c = a ^ b
    s = 1
    p = max(a, b)
    d = bin(a).count('1')
    e = 1
    while e <= p:
        if e & a:
            d -= 1
        if d & 1 and e & b:
            s = -s
        e *= 2
    return (c, s)


def _tables():
    dim, out = _DIM, _OUT
    rs = np.zeros((dim, dim), dtype=np.float32)
    for i in range(dim):
        for j in range(dim):
            _, s = _blade_combine(i, j)
            rs[i, j] = s
    ts = np.zeros((dim, dim, out, dim), dtype=np.float32)
    for i in range(dim):
        for j in range(dim):
            ts[j, i, :, i ^ j] = rs[i, j]
    ts = ts.reshape(dim, dim, out * dim)
    ed = np.kron(np.eye(out, dtype=np.float32), np.ones((1, dim), np.float32))
    return ts, ed


_TS, _ED = _tables()


def _body(x_ref, w_ref, ts_ref, ed_ref, b_ref, o_ref):
    wb = jax.lax.dot_general(
        w_ref[...], ed_ref[...], (((0,), (0,)), ((), ())),
        preferred_element_type=jnp.float32)
    g = jnp.sum(ts_ref[...] * wb[:, None, :], axis=0)
    acc = jax.lax.dot_general(
        x_ref[...], g, (((1,), (0,)), ((), ())),
        preferred_element_type=jnp.float32) + b_ref[...]
    o_ref[...] = acc.astype(jnp.bfloat16)


@functools.partial(jax.jit, static_argnames=())
def kernel(input, weight, bias):
    batch = input.shape[0]
    dim, out = _DIM, _OUT
    block = 4096
    bflat = jnp.tile(bias, out)[None, :]
    flat = pl.pallas_call(
        _body,
        grid=(batch // block,),
        in_specs=[
            pl.BlockSpec((block, dim), lambda i: (i, 0)),
            pl.BlockSpec((out, dim), lambda i: (0, 0)),
            pl.BlockSpec((dim, dim, out * dim), lambda i: (0, 0, 0)),
            pl.BlockSpec((out, out * dim), lambda i: (0, 0)),
            pl.BlockSpec((1, out * dim), lambda i: (0, 0)),
        ],
        out_specs=pl.BlockSpec((block, out * dim), lambda i: (i, 0)),
        out_shape=jax.ShapeDtypeStruct((batch, out * dim), jnp.bfloat16),
        compiler_params=pltpu.CompilerParams(
            dimension_semantics=("parallel",)),
    )(input, weight, jnp.asarray(_TS), jnp.asarray(_ED), bflat)
    return flat.astype(jnp.float32).reshape(batch, out, dim)
